# baseline (speedup 1.0000x reference)
import functools

import jax
import jax.numpy as jnp
from jax.experimental import pallas as pl
from jax.experimental.pallas import tpu as pltpu

_BN_EPS = 1e-5
_TAPS9 = tuple((dh, dw) for dh in range(3) for dw in range(3))
_PHASES = (
    ((0, 0), (((1, 1), (1, 1)),)),
    ((0, 1), (((1, 1), (1, 2)), ((1, 2), (1, 0)))),
    ((1, 0), (((1, 1), (2, 1)), ((2, 1), (0, 1)))),
    ((1, 1), (((1, 1), (2, 2)), ((1, 2), (2, 0)),
              ((2, 1), (0, 2)), ((2, 2), (0, 0)))),
)


def _bn_fold(b, gamma, beta, mean, var):
    scale = gamma * jax.lax.rsqrt(var + _BN_EPS)
    bias = (b - mean) * scale + beta
    return scale, bias


def _conv_w_taps(w_oihw, scale):
    w = jnp.transpose(w_oihw, (2, 3, 1, 0))
    w = w.reshape(9, w.shape[2], w.shape[3]) * scale[None, None, :]
    return w.astype(jnp.bfloat16)


def _convt_w_taps(w_iohw, scale):
    mats = []
    for _, group in _PHASES:
        for _, (kh, kw) in group:
            mats.append(w_iohw[:, :, kh, kw] * scale[None, :])
    return jnp.stack(mats, axis=0).astype(jnp.bfloat16)


def _apply_act(y, act):
    if act == "relu":
        return jnp.maximum(y, 0.0)
    if act == "tanh":
        return jnp.tanh(y)
    return y


def _conv_body(x_ref, w_ref, b_ref, o_ref, *, taps, stride, ho, wo, act):
    acc = None
    if stride == 2:
        cin = x_ref.shape[-1] // 2
        hp2 = x_ref.shape[1] // 2
        wp2 = x_ref.shape[2]
        x4 = x_ref[0].reshape(hp2, 2, wp2, 2 * cin)
    else:
        cin = x_ref.shape[-1]
    for t, (dh, dw) in enumerate(taps):
        if stride == 1:
            xt = x_ref[0, dh:dh + ho, dw:dw + wo, :]
        else:
            ph, oh = dh % 2, dh // 2
            pw, ow = dw % 2, dw // 2
            xt = x4[oh:oh + ho, ph, ow:ow + wo, pw * cin:(pw + 1) * cin]
        xt = xt.reshape(ho * wo, cin)
        d = jnp.dot(xt, w_ref[t], preferred_element_type=jnp.float32)
        acc = d if acc is None else acc + d
    y = _apply_act(acc + b_ref[...], act)
    co = o_ref.shape[3]
    o_ref[...] = y[:, :co].reshape(1, ho, wo, co).astype(o_ref.dtype)


def _conv_call(xp, w_taps, bias, *, stride, ho, wo, act, out_dtype,
               co=None, taps=_TAPS9):
    b_dim, hp, wp, cin = xp.shape
    t, _, co_w = w_taps.shape
    if co is None:
        co = co_w
    bias2 = bias.astype(jnp.float32).reshape(1, co_w)
    if stride == 2:
        xp = xp.reshape(b_dim, hp, wp // 2, 2 * cin)
        x_spec = pl.BlockSpec((1, hp, wp // 2, 2 * cin),
                              lambda b: (b, 0, 0, 0))
    else:
        x_spec = pl.BlockSpec((1, hp, wp, cin), lambda b: (b, 0, 0, 0))
    return pl.pallas_call(
        functools.partial(_conv_body, taps=taps, stride=stride, ho=ho,
                          wo=wo, act=act),
        out_shape=jax.ShapeDtypeStruct((b_dim, ho, wo, co), out_dtype),
        grid=(b_dim,),
        in_specs=[
            x_spec,
            pl.BlockSpec((t, w_taps.shape[1], co_w), lambda b: (0, 0, 0)),
            pl.BlockSpec((1, co_w), lambda b: (0, 0)),
        ],
        out_specs=pl.BlockSpec((1, ho, wo, co), lambda b: (b, 0, 0, 0)),
        compiler_params=pltpu.CompilerParams(
            dimension_semantics=("parallel",)),
    )(xp, w_taps, bias2)


def _convt_body(x_ref, w_ref, b_ref, o_ref, *, h, w):
    cin = x_ref.shape[3]
    co = o_ref.shape[5]
    idx = 0
    for p, (_, group) in enumerate(_PHASES):
        acc = None
        for (dh, dw), _ in group:
            xt = x_ref[0, dh:dh + h, dw:dw + w, :].reshape(h * w, cin)
            d = jnp.dot(xt, w_ref[idx], preferred_element_type=jnp.float32)
            acc = d if acc is None else acc + d
            idx += 1
        y = jnp.maximum(acc + b_ref[...], 0.0)
        pi, pj = _PHASES[p][0]
        o_ref[0, pi, pj] = y.reshape(h, w, co).astype(o_ref.dtype)


def _convt_call(xp, w_taps, bias, *, h, w):
    b_dim, hp, wp, cin = xp.shape
    t, _, co = w_taps.shape
    bias2 = bias.astype(jnp.float32).reshape(1, co)
    out = pl.pallas_call(
        functools.partial(_convt_body, h=h, w=w),
        out_shape=jax.ShapeDtypeStruct((b_dim, 2, 2, h, w, co),
                                       jnp.bfloat16),
        grid=(b_dim,),
        in_specs=[
            pl.BlockSpec((1, hp, wp, cin), lambda b: (b, 0, 0, 0)),
            pl.BlockSpec((t, cin, co), lambda b: (0, 0, 0)),
            pl.BlockSpec((1, co), lambda b: (0, 0)),
        ],
        out_specs=pl.BlockSpec((1, 2, 2, h, w, co),
                               lambda b: (b, 0, 0, 0, 0, 0)),
        compiler_params=pltpu.CompilerParams(
            dimension_semantics=("parallel",)),
    )(xp, w_taps, bias2)
    out = jnp.transpose(out, (0, 3, 1, 4, 2, 5))
    return out.reshape(b_dim, 2 * h, 2 * w, co)


def _ta_body(x_ref, w_ref, b_ref, o_ref, *, act):
    y = jax.lax.dot_general(x_ref[0], w_ref[...],
                            (((0,), (0,)), ((), ())),
                            preferred_element_type=jnp.float32)
    y = _apply_act(y + b_ref[...], act)
    o_ref[...] = y[None].astype(o_ref.dtype)


def _l1_body(x_ref, w_ref, b_ref, o_ref):
    m = 128 * 130
    rows = []
    for dh, dw in _TAPS9:
        s = dh * 130 + dw
        rows.append(x_ref[0, :, s:s + m])
    p = jnp.concatenate(rows, axis=0).astype(jnp.bfloat16)
    y = jax.lax.dot_general(p, w_ref[...], (((0,), (0,)), ((), ())),
                            preferred_element_type=jnp.float32)
    y = jnp.maximum(y + b_ref[...], 0.0)
    o_ref[...] = y[None].astype(jnp.bfloat16)


def _ta_call(patches, w_mat, bias, *, act):
    b_dim, k, m = patches.shape
    co = w_mat.shape[1]
    return pl.pallas_call(
        functools.partial(_ta_body, act=act),
        out_shape=jax.ShapeDtypeStruct((b_dim, m, co), jnp.bfloat16),
        grid=(b_dim,),
        in_specs=[
            pl.BlockSpec((1, k, m), lambda b: (b, 0, 0)),
            pl.BlockSpec((k, co), lambda b: (0, 0)),
            pl.BlockSpec((1, co), lambda b: (0, 0)),
        ],
        out_specs=pl.BlockSpec((1, m, co), lambda b: (b, 0, 0)),
        compiler_params=pltpu.CompilerParams(
            dimension_semantics=("parallel",)),
    )(patches, w_mat, bias.astype(jnp.float32).reshape(1, co))


def _nchw_patches(xp, stride, ho, wo):
    b_dim, c, hp, wp = xp.shape
    cols = [xp[:, :, dh:dh + stride * (ho - 1) + 1:stride,
               dw:dw + stride * (wo - 1) + 1:stride].reshape(b_dim, c, ho * wo)
            for (dh, dw) in _TAPS9]
    return jnp.concatenate(cols, axis=1).astype(jnp.bfloat16)


def _flow_body(x_ref, w_ref, b_ref, wa_ref, ba_ref, o_ref, a_ref, scr):
    acc = None
    for t, (dh, dw) in enumerate(_TAPS9):
        xt = x_ref[0, dh:dh + 128, dw:dw + 128, :].reshape(128 * 128, 64)
        d = jnp.dot(xt, w_ref[t], preferred_element_type=jnp.float32)
        acc = d if acc is None else acc + d
    y = jnp.tanh(acc + b_ref[...])
    o_ref[...] = jnp.transpose(y[:, :2])[None]
    scr[0:1] = jnp.zeros((1, 130, 128), jnp.float32)
    scr[129:130] = jnp.zeros((1, 130, 128), jnp.float32)
    scr[:, 0:1] = jnp.zeros((130, 1, 128), jnp.float32)
    scr[:, 129:130] = jnp.zeros((130, 1, 128), jnp.float32)
    scr[1:129, 1:129, :] = y.reshape(128, 128, 128)
    acc2 = None
    for t, (dh, dw) in enumerate(_TAPS9):
        xt = scr[dh:dh + 127:2, dw:dw + 127:2, :]
        xt2 = xt.reshape(4096, 128).astype(jnp.bfloat16)
        d = jnp.dot(xt2, wa_ref[t], preferred_element_type=jnp.float32)
        acc2 = d if acc2 is None else acc2 + d
    a = jnp.maximum(acc2 + ba_ref[...], 0.0)
    a_ref[...] = a.reshape(1, 64, 64, 64).astype(jnp.bfloat16)


def _warp_body(f_ref, a_ref, o_ref):
    dx = f_ref[0, 0]
    dy = f_ref[0, 1]
    zero = jnp.zeros_like(dx)
    cw = (jnp.where(dx < 0, -dx, zero),
          jnp.where(dx < 0, 1.0 + dx, 1.0 - dx),
          jnp.where(dx < 0, zero, dx))
    rw = (jnp.where(dy < 0, -dy, zero),
          jnp.where(dy < 0, 1.0 + dy, 1.0 - dy),
          jnp.where(dy < 0, zero, dy))
    for c in range(3):
        acc = None
        for i in range(3):
            for j in range(3):
                tap = a_ref[0, c, i:i + 128, j:j + 128] * (rw[i] * cw[j])
                acc = tap if acc is None else acc + tap
        o_ref[0, c] = acc


def _warp_nchw(anchor, flow):
    b_dim = anchor.shape[0]
    anc_p = jnp.pad(anchor, ((0, 0), (0, 0), (1, 1), (1, 1)))
    return pl.pallas_call(
        _warp_body,
        out_shape=jax.ShapeDtypeStruct((b_dim, 3, 128, 128), jnp.float32),
        grid=(b_dim,),
        in_specs=[
            pl.BlockSpec((1, 2, 128, 128), lambda b: (b, 0, 0, 0)),
            pl.BlockSpec((1, 3, 130, 130), lambda b: (b, 0, 0, 0)),
        ],
        out_specs=pl.BlockSpec((1, 3, 128, 128), lambda b: (b, 0, 0, 0)),
        compiler_params=pltpu.CompilerParams(
            dimension_semantics=("parallel",)),
    )(flow, anc_p)


def _fc_body(x_ref, w1_ref, b1_ref, w2_ref, b2_ref, o_ref):
    y = jnp.dot(x_ref[...], w1_ref[...], preferred_element_type=jnp.float32)
    y = jnp.maximum(y + b1_ref[...], 0.0).astype(jnp.bfloat16)
    z = jnp.dot(y, w2_ref[...], preferred_element_type=jnp.float32)
    o_ref[...] = z + b2_ref[...]


def _pad1(x):
    return jnp.pad(x, ((0, 0), (1, 1), (1, 1), (0, 0)))


def _im2col(xp, stride, ho, wo):
    cols = [xp[:, dh:dh + stride * (ho - 1) + 1:stride,
               dw:dw + stride * (wo - 1) + 1:stride, :]
            for (dh, dw) in _TAPS9]
    return jnp.concatenate(cols, axis=-1)


def kernel(image, anchor, of_l1_w, of_l1_b, of_l1_gamma, of_l1_beta, of_l1_mean, of_l1_var, of_l2_w, of_l2_b, of_l2_gamma, of_l2_beta, of_l2_mean, of_l2_var, of_l3_w, of_l3_b, of_l3_gamma, of_l3_beta, of_l3_mean, of_l3_var, of_l4_w, of_l4_b, of_l4_gamma, of_l4_beta, of_l4_mean, of_l4_var, of_l5_w, of_l5_b, of_l5_gamma, of_l5_beta, of_l5_mean, of_l5_var, of_l6_w, of_l6_b, of_l6_gamma, of_l6_beta, of_l6_mean, of_l6_var, of_l7_w, of_l7_b, of_l7_gamma, of_l7_beta, of_l7_mean, of_l7_var, of_l8_w, of_l8_b, of_l8_gamma, of_l8_beta, of_l8_mean, of_l8_var, of_l9_w, of_l9_b, of_l9_gamma, of_l9_beta, of_l9_mean, of_l9_var, of_l10_w, of_l10_b, of_l10_gamma, of_l10_beta, of_l10_mean, of_l10_var, au_l1_w, au_l1_b, au_l1_gamma, au_l1_beta, au_l1_mean, au_l1_var, au_l2_w, au_l2_b, au_l2_gamma, au_l2_beta, au_l2_mean, au_l2_var, au_l3_w, au_l3_b, au_l3_gamma, au_l3_beta, au_l3_mean, au_l3_var, au_l4_w, au_l4_b, au_l4_gamma, au_l4_beta, au_l4_mean, au_l4_var, au_l5_w, au_l5_b, au_l5_gamma, au_l5_beta, au_l5_mean, au_l5_var, au_fc1_w, au_fc1_b, au_fc2_w, au_fc2_b):
    B = image.shape[0]

    def conv_params(w, b, g, be, m, v, transpose=False):
        scale, bias = _bn_fold(b, g, be, m, v)
        if transpose:
            return _convt_w_taps(w, scale), bias
        return _conv_w_taps(w, scale), bias

    w1, b1 = conv_params(of_l1_w, of_l1_b, of_l1_gamma, of_l1_beta,
                         of_l1_mean, of_l1_var)
    imgf = jnp.pad(jnp.pad(image, ((0, 0), (0, 0), (1, 1), (1, 1))).reshape(
        B, 3, 130 * 130), ((0, 0), (0, 0), (0, 4)))
    xw = pl.pallas_call(
        _l1_body,
        out_shape=jax.ShapeDtypeStruct((B, 128 * 130, 64), jnp.bfloat16),
        grid=(B,),
        in_specs=[
            pl.BlockSpec((1, 3, 130 * 130 + 4), lambda b: (b, 0, 0)),
            pl.BlockSpec((27, 64), lambda b: (0, 0)),
            pl.BlockSpec((1, 64), lambda b: (0, 0)),
        ],
        out_specs=pl.BlockSpec((1, 128 * 130, 64), lambda b: (b, 0, 0)),
        compiler_params=pltpu.CompilerParams(
            dimension_semantics=("parallel",)),
    )(imgf, w1.reshape(27, 64),
      b1.astype(jnp.float32).reshape(1, 64))
    x = xw.reshape(B, 128, 130, 64)[:, :, :128, :]
    for (wp, bp, gp, bep, mp, vp, ho) in (
            (of_l2_w, of_l2_b, of_l2_gamma, of_l2_beta, of_l2_mean,
             of_l2_var, 64),
            (of_l3_w, of_l3_b, of_l3_gamma, of_l3_beta, of_l3_mean,
             of_l3_var, 32),
            (of_l4_w, of_l4_b, of_l4_gamma, of_l4_beta, of_l4_mean,
             of_l4_var, 16),
            (of_l5_w, of_l5_b, of_l5_gamma, of_l5_beta, of_l5_mean,
             of_l5_var, 8)):
        wt, bt = conv_params(wp, bp, gp, bep, mp, vp)
        x = _conv_call(_pad1(x), wt, bt, stride=2, ho=ho, wo=ho,
                       act="relu", out_dtype=jnp.bfloat16)

    for (wp, bp, gp, bep, mp, vp, h) in (
            (of_l6_w, of_l6_b, of_l6_gamma, of_l6_beta, of_l6_mean,
             of_l6_var, 8),
            (of_l7_w, of_l7_b, of_l7_gamma, of_l7_beta, of_l7_mean,
             of_l7_var, 16),
            (of_l8_w, of_l8_b, of_l8_gamma, of_l8_beta, of_l8_mean,
             of_l8_var, 32),
            (of_l9_w, of_l9_b, of_l9_gamma, of_l9_beta, of_l9_mean,
             of_l9_var, 64)):
        wt, bt = conv_params(wp, bp, gp, bep, mp, vp, transpose=True)
        x = _convt_call(_pad1(x), wt, bt, h=h, w=h)

    w10 = jnp.transpose(of_l10_w[:, :, ::-1, ::-1], (1, 0, 2, 3))
    w10t = jnp.pad(_conv_w_taps(w10, jnp.ones_like(of_l10_b)).astype(
        jnp.float32), ((0, 0), (0, 0), (0, 126))).astype(jnp.bfloat16)
    b10 = jnp.pad(of_l10_b, (0, 126)).astype(jnp.float32).reshape(1, 128)
    wa1, ba1 = conv_params(au_l1_w, au_l1_b, au_l1_gamma, au_l1_beta,
                           au_l1_mean, au_l1_var)
    wa1 = jnp.pad(wa1.astype(jnp.float32),
                  ((0, 0), (0, 126), (0, 0))).astype(jnp.bfloat16)
    flow_cm, a = pl.pallas_call(
        _flow_body,
        out_shape=(jax.ShapeDtypeStruct((B, 2, 128 * 128), jnp.float32),
                   jax.ShapeDtypeStruct((B, 64, 64, 64), jnp.bfloat16)),
        grid=(B,),
        in_specs=[
            pl.BlockSpec((1, 130, 130, 64), lambda b: (b, 0, 0, 0)),
            pl.BlockSpec((9, 64, 128), lambda b: (0, 0, 0)),
            pl.BlockSpec((1, 128), lambda b: (0, 0)),
            pl.BlockSpec((9, 128, 64), lambda b: (0, 0, 0)),
            pl.BlockSpec((1, 64), lambda b: (0, 0)),
        ],
        out_specs=(pl.BlockSpec((1, 2, 128 * 128), lambda b: (b, 0, 0)),
                   pl.BlockSpec((1, 64, 64, 64), lambda b: (b, 0, 0, 0))),
        scratch_shapes=[pltpu.VMEM((130, 130, 128), jnp.float32)],
        compiler_params=pltpu.CompilerParams(
            dimension_semantics=("parallel",)),
    )(_pad1(x), w10t, b10, wa1,
      ba1.astype(jnp.float32).reshape(1, 64))
    flow = flow_cm.reshape(B, 2, 128, 128)

    recon = _warp_nchw(anchor, flow)

    for (wp, bp, gp, bep, mp, vp, ho) in (
            (au_l2_w, au_l2_b, au_l2_gamma, au_l2_beta, au_l2_mean,
             au_l2_var, 32),
            (au_l3_w, au_l3_b, au_l3_gamma, au_l3_beta, au_l3_mean,
             au_l3_var, 16),
            (au_l4_w, au_l4_b, au_l4_gamma, au_l4_beta, au_l4_mean,
             au_l4_var, 8)):
        wt, bt = conv_params(wp, bp, gp, bep, mp, vp)
        a = _conv_call(_pad1(a), wt, bt, stride=2, ho=ho, wo=ho,
                       act="relu", out_dtype=jnp.bfloat16)
    wa5, ba5 = conv_params(au_l5_w, au_l5_b, au_l5_gamma, au_l5_beta,
                           au_l5_mean, au_l5_var)
    p5 = _im2col(_pad1(a), 2, 4, 4).reshape(B, 16, 9 * 512)
    a5 = _conv_call(p5.reshape(B, 4, 4, 9 * 512),
                    wa5.reshape(1, 9 * 512, 512), ba5, stride=1, ho=4,
                    wo=4, act="relu", out_dtype=jnp.bfloat16,
                    taps=((0, 0),))

    xf = jnp.transpose(a5, (0, 3, 1, 2)).reshape(B, 512 * 16)
    w_fc1 = au_fc1_w.T.astype(jnp.bfloat16)
    w_fc2 = jnp.pad(au_fc2_w.T, ((0, 0), (0, 128 - 12))).astype(jnp.bfloat16)
    b_fc2 = jnp.pad(au_fc2_b, (0, 128 - 12))
    au_label = pl.pallas_call(
        _fc_body,
        out_shape=jax.ShapeDtypeStruct((B, 128), jnp.float32),
        grid=(1,),
        in_specs=[
            pl.BlockSpec((B, 8192), lambda i: (0, 0)),
            pl.BlockSpec((8192, 1024), lambda i: (0, 0)),
            pl.BlockSpec((1, 1024), lambda i: (0, 0)),
            pl.BlockSpec((1024, 128), lambda i: (0, 0)),
            pl.BlockSpec((1, 128), lambda i: (0, 0)),
        ],
        out_specs=pl.BlockSpec((B, 128), lambda i: (0, 0)),
        compiler_params=pltpu.CompilerParams(
            dimension_semantics=("arbitrary",)),
    )(xf, w_fc1, au_fc1_b.astype(jnp.float32).reshape(1, 1024), w_fc2,
      b_fc2.astype(jnp.float32).reshape(1, 128))[:, :12]

    return au_label, recon, flow


# speedup vs baseline: 1.0415x; 1.0415x over previous
import functools

import jax
import jax.numpy as jnp
from jax.experimental import pallas as pl
from jax.experimental.pallas import tpu as pltpu

_BN_EPS = 1e-5
_TAPS9 = tuple((dh, dw) for dh in range(3) for dw in range(3))
_PHASES = (
    ((0, 0), (((1, 1), (1, 1)),)),
    ((0, 1), (((1, 1), (1, 2)), ((1, 2), (1, 0)))),
    ((1, 0), (((1, 1), (2, 1)), ((2, 1), (0, 1)))),
    ((1, 1), (((1, 1), (2, 2)), ((1, 2), (2, 0)),
              ((2, 1), (0, 2)), ((2, 2), (0, 0)))),
)


def _bn_fold(b, gamma, beta, mean, var):
    scale = gamma * jax.lax.rsqrt(var + _BN_EPS)
    bias = (b - mean) * scale + beta
    return scale, bias


def _conv_w_taps(w_oihw, scale):
    w = jnp.transpose(w_oihw, (2, 3, 1, 0))
    w = w.reshape(9, w.shape[2], w.shape[3]) * scale[None, None, :]
    return w.astype(jnp.bfloat16)


def _convt_w_taps(w_iohw, scale):
    mats = []
    for _, group in _PHASES:
        for _, (kh, kw) in group:
            mats.append(w_iohw[:, :, kh, kw] * scale[None, :])
    return jnp.stack(mats, axis=0).astype(jnp.bfloat16)


def _apply_act(y, act):
    if act == "relu":
        return jnp.maximum(y, 0.0)
    if act == "tanh":
        return jnp.tanh(y)
    return y


def _conv_body(x_ref, w_ref, b_ref, o_ref, *, taps, stride, ho, wo, act):
    acc = None
    if stride == 2:
        cin = x_ref.shape[-1] // 2
        hp2 = x_ref.shape[1] // 2
        wp2 = x_ref.shape[2]
        x4 = x_ref[0].reshape(hp2, 2, wp2, 2 * cin)
    else:
        cin = x_ref.shape[-1]
    for t, (dh, dw) in enumerate(taps):
        if stride == 1:
            xt = x_ref[0, dh:dh + ho, dw:dw + wo, :]
        else:
            ph, oh = dh % 2, dh // 2
            pw, ow = dw % 2, dw // 2
            xt = x4[oh:oh + ho, ph, ow:ow + wo, pw * cin:(pw + 1) * cin]
        xt = xt.reshape(ho * wo, cin)
        d = jnp.dot(xt, w_ref[t], preferred_element_type=jnp.float32)
        acc = d if acc is None else acc + d
    y = _apply_act(acc + b_ref[...], act)
    co = o_ref.shape[3]
    o_ref[...] = y[:, :co].reshape(1, ho, wo, co).astype(o_ref.dtype)


def _conv_call(xp, w_taps, bias, *, stride, ho, wo, act, out_dtype,
               co=None, taps=_TAPS9):
    b_dim, hp, wp, cin = xp.shape
    t, _, co_w = w_taps.shape
    if co is None:
        co = co_w
    bias2 = bias.astype(jnp.float32).reshape(1, co_w)
    if stride == 2:
        xp = xp.reshape(b_dim, hp, wp // 2, 2 * cin)
        x_spec = pl.BlockSpec((1, hp, wp // 2, 2 * cin),
                              lambda b: (b, 0, 0, 0))
    else:
        x_spec = pl.BlockSpec((1, hp, wp, cin), lambda b: (b, 0, 0, 0))
    return pl.pallas_call(
        functools.partial(_conv_body, taps=taps, stride=stride, ho=ho,
                          wo=wo, act=act),
        out_shape=jax.ShapeDtypeStruct((b_dim, ho, wo, co), out_dtype),
        grid=(b_dim,),
        in_specs=[
            x_spec,
            pl.BlockSpec((t, w_taps.shape[1], co_w), lambda b: (0, 0, 0)),
            pl.BlockSpec((1, co_w), lambda b: (0, 0)),
        ],
        out_specs=pl.BlockSpec((1, ho, wo, co), lambda b: (b, 0, 0, 0)),
        compiler_params=pltpu.CompilerParams(
            dimension_semantics=("parallel",)),
    )(xp, w_taps, bias2)


def _convt_body(x_ref, w_ref, b_ref, o_ref, *, h, w):
    cin = x_ref.shape[3]
    co = o_ref.shape[5]
    idx = 0
    for p, (_, group) in enumerate(_PHASES):
        acc = None
        for (dh, dw), _ in group:
            xt = x_ref[0, dh:dh + h, dw:dw + w, :].reshape(h * w, cin)
            d = jnp.dot(xt, w_ref[idx], preferred_element_type=jnp.float32)
            acc = d if acc is None else acc + d
            idx += 1
        y = jnp.maximum(acc + b_ref[...], 0.0)
        pi, pj = _PHASES[p][0]
        o_ref[0, pi, pj] = y.reshape(h, w, co).astype(o_ref.dtype)


def _convt_call(xp, w_taps, bias, *, h, w):
    b_dim, hp, wp, cin = xp.shape
    t, _, co = w_taps.shape
    bias2 = bias.astype(jnp.float32).reshape(1, co)
    out = pl.pallas_call(
        functools.partial(_convt_body, h=h, w=w),
        out_shape=jax.ShapeDtypeStruct((b_dim, 2, 2, h, w, co),
                                       jnp.bfloat16),
        grid=(b_dim,),
        in_specs=[
            pl.BlockSpec((1, hp, wp, cin), lambda b: (b, 0, 0, 0)),
            pl.BlockSpec((t, cin, co), lambda b: (0, 0, 0)),
            pl.BlockSpec((1, co), lambda b: (0, 0)),
        ],
        out_specs=pl.BlockSpec((1, 2, 2, h, w, co),
                               lambda b: (b, 0, 0, 0, 0, 0)),
        compiler_params=pltpu.CompilerParams(
            dimension_semantics=("parallel",)),
    )(xp, w_taps, bias2)
    out = jnp.transpose(out, (0, 3, 1, 4, 2, 5))
    return out.reshape(b_dim, 2 * h, 2 * w, co)


def _ta_body(x_ref, w_ref, b_ref, o_ref, *, act):
    y = jax.lax.dot_general(x_ref[0], w_ref[...],
                            (((0,), (0,)), ((), ())),
                            preferred_element_type=jnp.float32)
    y = _apply_act(y + b_ref[...], act)
    o_ref[...] = y[None].astype(o_ref.dtype)


def _ta_call(patches, w_mat, bias, *, act):
    b_dim, k, m = patches.shape
    co = w_mat.shape[1]
    return pl.pallas_call(
        functools.partial(_ta_body, act=act),
        out_shape=jax.ShapeDtypeStruct((b_dim, m, co), jnp.bfloat16),
        grid=(b_dim,),
        in_specs=[
            pl.BlockSpec((1, k, m), lambda b: (b, 0, 0)),
            pl.BlockSpec((k, co), lambda b: (0, 0)),
            pl.BlockSpec((1, co), lambda b: (0, 0)),
        ],
        out_specs=pl.BlockSpec((1, m, co), lambda b: (b, 0, 0)),
        compiler_params=pltpu.CompilerParams(
            dimension_semantics=("parallel",)),
    )(patches, w_mat, bias.astype(jnp.float32).reshape(1, co))


def _nchw_patches(xp, stride, ho, wo):
    b_dim, c, hp, wp = xp.shape
    cols = [xp[:, :, dh:dh + stride * (ho - 1) + 1:stride,
               dw:dw + stride * (wo - 1) + 1:stride].reshape(b_dim, c, ho * wo)
            for (dh, dw) in _TAPS9]
    return jnp.concatenate(cols, axis=1).astype(jnp.bfloat16)


def _flow_body(x_ref, w_ref, b_ref, wa_ref, ba_ref, o_ref, a_ref, scr):
    acc = None
    for t, (dh, dw) in enumerate(_TAPS9):
        xt = x_ref[0, dh:dh + 128, dw:dw + 128, :].reshape(128 * 128, 64)
        d = jnp.dot(xt, w_ref[t], preferred_element_type=jnp.float32)
        acc = d if acc is None else acc + d
    y = jnp.tanh(acc + b_ref[...])
    o_ref[...] = jnp.transpose(y[:, :2])[None]
    scr[0:1] = jnp.zeros((1, 130, 128), jnp.float32)
    scr[129:130] = jnp.zeros((1, 130, 128), jnp.float32)
    scr[:, 0:1] = jnp.zeros((130, 1, 128), jnp.float32)
    scr[:, 129:130] = jnp.zeros((130, 1, 128), jnp.float32)
    scr[1:129, 1:129, :] = y.reshape(128, 128, 128)
    acc2 = None
    for t, (dh, dw) in enumerate(_TAPS9):
        xt = scr[dh:dh + 127:2, dw:dw + 127:2, :]
        xt2 = xt.reshape(4096, 128).astype(jnp.bfloat16)
        d = jnp.dot(xt2, wa_ref[t], preferred_element_type=jnp.float32)
        acc2 = d if acc2 is None else acc2 + d
    a = jnp.maximum(acc2 + ba_ref[...], 0.0)
    a_ref[...] = a.reshape(1, 64, 64, 64).astype(jnp.bfloat16)


def _warp_body(f_ref, a_ref, o_ref):
    dx = f_ref[0, 0]
    dy = f_ref[0, 1]
    zero = jnp.zeros_like(dx)
    cw = (jnp.where(dx < 0, -dx, zero),
          jnp.where(dx < 0, 1.0 + dx, 1.0 - dx),
          jnp.where(dx < 0, zero, dx))
    rw = (jnp.where(dy < 0, -dy, zero),
          jnp.where(dy < 0, 1.0 + dy, 1.0 - dy),
          jnp.where(dy < 0, zero, dy))
    for c in range(3):
        acc = None
        for i in range(3):
            for j in range(3):
                tap = a_ref[0, c, i:i + 128, j:j + 128] * (rw[i] * cw[j])
                acc = tap if acc is None else acc + tap
        o_ref[0, c] = acc


def _warp_nchw(anchor, flow):
    b_dim = anchor.shape[0]
    anc_p = jnp.pad(anchor, ((0, 0), (0, 0), (1, 1), (1, 1)))
    return pl.pallas_call(
        _warp_body,
        out_shape=jax.ShapeDtypeStruct((b_dim, 3, 128, 128), jnp.float32),
        grid=(b_dim,),
        in_specs=[
            pl.BlockSpec((1, 2, 128, 128), lambda b: (b, 0, 0, 0)),
            pl.BlockSpec((1, 3, 130, 130), lambda b: (b, 0, 0, 0)),
        ],
        out_specs=pl.BlockSpec((1, 3, 128, 128), lambda b: (b, 0, 0, 0)),
        compiler_params=pltpu.CompilerParams(
            dimension_semantics=("parallel",)),
    )(flow, anc_p)


def _fc_body(x_ref, w1_ref, b1_ref, w2_ref, b2_ref, o_ref):
    y = jnp.dot(x_ref[...], w1_ref[...], preferred_element_type=jnp.float32)
    y = jnp.maximum(y + b1_ref[...], 0.0).astype(jnp.bfloat16)
    z = jnp.dot(y, w2_ref[...], preferred_element_type=jnp.float32)
    o_ref[...] = z + b2_ref[...]


def _pad1(x):
    return jnp.pad(x, ((0, 0), (1, 1), (1, 1), (0, 0)))


def _im2col(xp, stride, ho, wo):
    cols = [xp[:, dh:dh + stride * (ho - 1) + 1:stride,
               dw:dw + stride * (wo - 1) + 1:stride, :]
            for (dh, dw) in _TAPS9]
    return jnp.concatenate(cols, axis=-1)


def kernel(image, anchor, of_l1_w, of_l1_b, of_l1_gamma, of_l1_beta, of_l1_mean, of_l1_var, of_l2_w, of_l2_b, of_l2_gamma, of_l2_beta, of_l2_mean, of_l2_var, of_l3_w, of_l3_b, of_l3_gamma, of_l3_beta, of_l3_mean, of_l3_var, of_l4_w, of_l4_b, of_l4_gamma, of_l4_beta, of_l4_mean, of_l4_var, of_l5_w, of_l5_b, of_l5_gamma, of_l5_beta, of_l5_mean, of_l5_var, of_l6_w, of_l6_b, of_l6_gamma, of_l6_beta, of_l6_mean, of_l6_var, of_l7_w, of_l7_b, of_l7_gamma, of_l7_beta, of_l7_mean, of_l7_var, of_l8_w, of_l8_b, of_l8_gamma, of_l8_beta, of_l8_mean, of_l8_var, of_l9_w, of_l9_b, of_l9_gamma, of_l9_beta, of_l9_mean, of_l9_var, of_l10_w, of_l10_b, of_l10_gamma, of_l10_beta, of_l10_mean, of_l10_var, au_l1_w, au_l1_b, au_l1_gamma, au_l1_beta, au_l1_mean, au_l1_var, au_l2_w, au_l2_b, au_l2_gamma, au_l2_beta, au_l2_mean, au_l2_var, au_l3_w, au_l3_b, au_l3_gamma, au_l3_beta, au_l3_mean, au_l3_var, au_l4_w, au_l4_b, au_l4_gamma, au_l4_beta, au_l4_mean, au_l4_var, au_l5_w, au_l5_b, au_l5_gamma, au_l5_beta, au_l5_mean, au_l5_var, au_fc1_w, au_fc1_b, au_fc2_w, au_fc2_b):
    B = image.shape[0]

    def conv_params(w, b, g, be, m, v, transpose=False):
        scale, bias = _bn_fold(b, g, be, m, v)
        if transpose:
            return _convt_w_taps(w, scale), bias
        return _conv_w_taps(w, scale), bias

    w1, b1 = conv_params(of_l1_w, of_l1_b, of_l1_gamma, of_l1_beta,
                         of_l1_mean, of_l1_var)
    imgp = jnp.pad(image, ((0, 0), (0, 0), (1, 1), (1, 1)))
    p1 = _nchw_patches(imgp, 1, 128, 128)
    x = _ta_call(p1, w1.reshape(27, 64), b1, act="relu")
    x = x.reshape(B, 128, 128, 64)
    for (wp, bp, gp, bep, mp, vp, ho) in (
            (of_l2_w, of_l2_b, of_l2_gamma, of_l2_beta, of_l2_mean,
             of_l2_var, 64),
            (of_l3_w, of_l3_b, of_l3_gamma, of_l3_beta, of_l3_mean,
             of_l3_var, 32),
            (of_l4_w, of_l4_b, of_l4_gamma, of_l4_beta, of_l4_mean,
             of_l4_var, 16),
            (of_l5_w, of_l5_b, of_l5_gamma, of_l5_beta, of_l5_mean,
             of_l5_var, 8)):
        wt, bt = conv_params(wp, bp, gp, bep, mp, vp)
        x = _conv_call(_pad1(x), wt, bt, stride=2, ho=ho, wo=ho,
                       act="relu", out_dtype=jnp.bfloat16)

    for (wp, bp, gp, bep, mp, vp, h) in (
            (of_l6_w, of_l6_b, of_l6_gamma, of_l6_beta, of_l6_mean,
             of_l6_var, 8),
            (of_l7_w, of_l7_b, of_l7_gamma, of_l7_beta, of_l7_mean,
             of_l7_var, 16),
            (of_l8_w, of_l8_b, of_l8_gamma, of_l8_beta, of_l8_mean,
             of_l8_var, 32),
            (of_l9_w, of_l9_b, of_l9_gamma, of_l9_beta, of_l9_mean,
             of_l9_var, 64)):
        wt, bt = conv_params(wp, bp, gp, bep, mp, vp, transpose=True)
        x = _convt_call(_pad1(x), wt, bt, h=h, w=h)

    w10 = jnp.transpose(of_l10_w[:, :, ::-1, ::-1], (1, 0, 2, 3))
    w10t = jnp.pad(_conv_w_taps(w10, jnp.ones_like(of_l10_b)).astype(
        jnp.float32), ((0, 0), (0, 0), (0, 126))).astype(jnp.bfloat16)
    b10 = jnp.pad(of_l10_b, (0, 126)).astype(jnp.float32).reshape(1, 128)
    wa1, ba1 = conv_params(au_l1_w, au_l1_b, au_l1_gamma, au_l1_beta,
                           au_l1_mean, au_l1_var)
    wa1 = jnp.pad(wa1.astype(jnp.float32),
                  ((0, 0), (0, 126), (0, 0))).astype(jnp.bfloat16)
    flow_cm, a = pl.pallas_call(
        _flow_body,
        out_shape=(jax.ShapeDtypeStruct((B, 2, 128 * 128), jnp.float32),
                   jax.ShapeDtypeStruct((B, 64, 64, 64), jnp.bfloat16)),
        grid=(B,),
        in_specs=[
            pl.BlockSpec((1, 130, 130, 64), lambda b: (b, 0, 0, 0)),
            pl.BlockSpec((9, 64, 128), lambda b: (0, 0, 0)),
            pl.BlockSpec((1, 128), lambda b: (0, 0)),
            pl.BlockSpec((9, 128, 64), lambda b: (0, 0, 0)),
            pl.BlockSpec((1, 64), lambda b: (0, 0)),
        ],
        out_specs=(pl.BlockSpec((1, 2, 128 * 128), lambda b: (b, 0, 0)),
                   pl.BlockSpec((1, 64, 64, 64), lambda b: (b, 0, 0, 0))),
        scratch_shapes=[pltpu.VMEM((130, 130, 128), jnp.float32)],
        compiler_params=pltpu.CompilerParams(
            dimension_semantics=("parallel",)),
    )(_pad1(x), w10t, b10, wa1,
      ba1.astype(jnp.float32).reshape(1, 64))
    flow = flow_cm.reshape(B, 2, 128, 128)

    recon = _warp_nchw(anchor, flow)

    for (wp, bp, gp, bep, mp, vp, ho) in (
            (au_l2_w, au_l2_b, au_l2_gamma, au_l2_beta, au_l2_mean,
             au_l2_var, 32),
            (au_l3_w, au_l3_b, au_l3_gamma, au_l3_beta, au_l3_mean,
             au_l3_var, 16),
            (au_l4_w, au_l4_b, au_l4_gamma, au_l4_beta, au_l4_mean,
             au_l4_var, 8)):
        wt, bt = conv_params(wp, bp, gp, bep, mp, vp)
        a = _conv_call(_pad1(a), wt, bt, stride=2, ho=ho, wo=ho,
                       act="relu", out_dtype=jnp.bfloat16)
    wa5, ba5 = conv_params(au_l5_w, au_l5_b, au_l5_gamma, au_l5_beta,
                           au_l5_mean, au_l5_var)
    p5 = _im2col(_pad1(a), 2, 4, 4).reshape(B, 16, 9 * 512)
    a5 = _conv_call(p5.reshape(B, 4, 4, 9 * 512),
                    wa5.reshape(1, 9 * 512, 512), ba5, stride=1, ho=4,
                    wo=4, act="relu", out_dtype=jnp.bfloat16,
                    taps=((0, 0),))

    xf = jnp.transpose(a5, (0, 3, 1, 2)).reshape(B, 512 * 16)
    w_fc1 = au_fc1_w.T.astype(jnp.bfloat16)
    w_fc2 = jnp.pad(au_fc2_w.T, ((0, 0), (0, 128 - 12))).astype(jnp.bfloat16)
    b_fc2 = jnp.pad(au_fc2_b, (0, 128 - 12))
    au_label = pl.pallas_call(
        _fc_body,
        out_shape=jax.ShapeDtypeStruct((B, 128), jnp.float32),
        grid=(1,),
        in_specs=[
            pl.BlockSpec((B, 8192), lambda i: (0, 0)),
            pl.BlockSpec((8192, 1024), lambda i: (0, 0)),
            pl.BlockSpec((1, 1024), lambda i: (0, 0)),
            pl.BlockSpec((1024, 128), lambda i: (0, 0)),
            pl.BlockSpec((1, 128), lambda i: (0, 0)),
        ],
        out_specs=pl.BlockSpec((B, 128), lambda i: (0, 0)),
        compiler_params=pltpu.CompilerParams(
            dimension_semantics=("arbitrary",)),
    )(xf, w_fc1, au_fc1_b.astype(jnp.float32).reshape(1, 1024), w_fc2,
      b_fc2.astype(jnp.float32).reshape(1, 128))[:, :12]

    return au_label, recon, flow
